# baseline (speedup 1.0000x reference)
import functools

import jax
import jax.numpy as jnp
from jax import lax
from jax.experimental import pallas as pl
from jax.experimental.pallas import tpu as pltpu
from jax.experimental.pallas import tpu_sc as plsc

N = 10000
E = 160000
D = 256
H = 256
C = 40
Q = 64

NC, NS = 2, 16
NW = NC * NS
NPAD = 10240
RPT = NPAD // NS
B = 128
CH12 = 80
CH3 = 40
BR = 1000

_SC_PARAMS = pltpu.CompilerParams(needs_layout_passes=False,
                                  use_tc_tiling_on_sc=False)


def _sc_mesh():
    return plsc.VectorSubcoreMesh(core_axis_name="c", subcore_axis_name="s")


@functools.partial(
    pl.kernel,
    out_type=jax.ShapeDtypeStruct((NC * NPAD,), jnp.float32),
    mesh=_sc_mesh(),
    compiler_params=_SC_PARAMS,
    scratch_types=[
        pltpu.VMEM((CH3, B), jnp.int32),
        pltpu.VMEM((NPAD,), jnp.float32),
        pltpu.VMEM((RPT,), jnp.float32),
        pltpu.VMEM((RPT,), jnp.float32),
        pltpu.VMEM_SHARED((NS * NPAD,), jnp.float32),
    ],
)
def _deg_kernel(dst_hbm, deg_out, slab, deg_v, tmp_v, acc_v, parts):
    c = lax.axis_index("c")
    s = lax.axis_index("s")
    w = c * NS + s
    zero16 = jnp.zeros((16,), jnp.float32)

    def zbody(i, _):
        deg_v[pl.ds(i * 16, 16)] = zero16
        return 0
    lax.fori_loop(0, NPAD // 16, zbody, 0)

    pltpu.sync_copy(dst_hbm.at[w], slab)
    ones = jnp.ones((16,), jnp.float32)

    def body(i, _):
        for k in range(B // 16):
            idx = slab[i, pl.ds(k * 16, 16)]
            plsc.addupdate_scatter(deg_v, [idx], ones)
        return 0
    lax.fori_loop(0, CH3, body, 0)

    pltpu.sync_copy(deg_v, parts.at[pl.ds(s * NPAD, NPAD)])
    plsc.subcore_barrier()

    def zbody2(i, _):
        acc_v[pl.ds(i * 16, 16)] = zero16
        return 0
    lax.fori_loop(0, RPT // 16, zbody2, 0)

    def rbody(j, _):
        pltpu.sync_copy(parts.at[pl.ds(j * NPAD + s * RPT, RPT)], tmp_v)

        def abody(k, _):
            acc_v[pl.ds(k * 16, 16)] = acc_v[pl.ds(k * 16, 16)] + tmp_v[pl.ds(k * 16, 16)]
            return 0
        lax.fori_loop(0, RPT // 16, abody, 0)
        return 0
    lax.fori_loop(0, NS, rbody, 0)

    pltpu.sync_copy(acc_v, deg_out.at[pl.ds(c * NPAD + s * RPT, RPT)])


def _make_agg(CH, npass):
    @functools.partial(
        pl.kernel,
        out_type=jax.ShapeDtypeStruct((2 * npass * NPAD, Q), jnp.float32),
        mesh=_sc_mesh(),
        compiler_params=_SC_PARAMS,
        scratch_types=[
            [pltpu.VMEM((CH, B), jnp.int32) for _ in range(npass)],
            pltpu.VMEM((CH, B), jnp.int32),
            [pltpu.VMEM((B, Q), jnp.float32) for _ in range(4)],
            pltpu.VMEM((B, Q), jnp.float32),
            pltpu.VMEM_SHARED((NPAD, Q), jnp.float32),
            [pltpu.SemaphoreType.DMA for _ in range(4)],
            [pltpu.SemaphoreType.DMA for _ in range(4)],
        ],
    )
    def agg(p_hbm, *args):
        src_hbms = args[:npass]
        dst_hbm = args[npass]
        out_hbm = args[npass + 1]
        src_vs = args[npass + 2]
        dst_v, rows, zb, acc, gsem, ssem = args[npass + 3:npass + 9]
        c = lax.axis_index("c")
        s = lax.axis_index("s")
        w = c * NS + s
        zero16 = jnp.zeros((16,), jnp.float32)

        def zrow(i, _):
            for k in range(Q // 16):
                zb[i, pl.ds(k * 16, 16)] = zero16
            return 0
        lax.fori_loop(0, B, zrow, 0)

        for k in range(npass):
            pltpu.sync_copy(src_hbms[k].at[w], src_vs[k])
        pltpu.sync_copy(dst_hbm.at[w], dst_v)

        for k in range(npass):
            src_v = src_vs[k]

            def zacc(j, _):
                pltpu.sync_copy(zb, acc.at[pl.ds(s * RPT + j * B, B)])
                return 0
            lax.fori_loop(0, RPT // B, zacc, 0)
            plsc.subcore_barrier()

            G = CH // 4
            for b in range(4):
                pltpu.async_copy(p_hbm.at[src_v.at[b]], rows[b], gsem[b])

            def step(g, _):
                ch = 4 * g
                for b in range(4):
                    pltpu.make_async_copy(p_hbm.at[src_v.at[ch + b]],
                                          rows[b], gsem[b]).wait()
                    pltpu.async_copy(rows[b], acc.at[dst_v.at[ch + b]],
                                     ssem[b], add=True)

                @pl.when(g < G - 1)
                def _():
                    for b in range(4):
                        pltpu.make_async_copy(rows[b],
                                              acc.at[dst_v.at[ch + b]],
                                              ssem[b]).wait()
                        pltpu.async_copy(p_hbm.at[src_v.at[ch + 4 + b]],
                                         rows[b], gsem[b])
                return 0
            lax.fori_loop(0, G, step, 0)

            for b in range(4):
                pltpu.make_async_copy(rows[b], acc.at[dst_v.at[CH - 4 + b]],
                                      ssem[b]).wait()

            plsc.subcore_barrier()
            pltpu.sync_copy(
                acc.at[pl.ds(s * RPT, RPT)],
                out_hbm.at[pl.ds((2 * k + c) * NPAD + s * RPT, RPT)])
            if k + 1 < npass:
                plsc.subcore_barrier()
    return agg


_agg12 = _make_agg(CH12, 2)
_agg3 = _make_agg(CH3, 1)


def _store_quarters(o_ref, p):
    for q in range(4):
        o_ref[q, :, :] = p[:, q * Q:(q + 1) * Q]


def _cat_quarters(s_ref):
    return jnp.concatenate([s_ref[q] for q in range(4)], axis=1)


def _tc1_body(x_ref, w_ref, da_ref, db_ref, dinv_ref, p_ref):
    d = lax.rsqrt(da_ref[...] + db_ref[...] + 1.0)
    dinv_ref[...] = d
    h = jnp.dot(x_ref[...], w_ref[...], preferred_element_type=jnp.float32)
    _store_quarters(p_ref, h * d)


def _tc1(x, W1, degA, degB):
    return pl.pallas_call(
        _tc1_body,
        grid=(N // BR,),
        in_specs=[
            pl.BlockSpec((BR, D), lambda r: (r, 0)),
            pl.BlockSpec((D, H), lambda r: (0, 0)),
            pl.BlockSpec((BR, 1), lambda r: (r, 0)),
            pl.BlockSpec((BR, 1), lambda r: (r, 0)),
        ],
        out_specs=[
            pl.BlockSpec((BR, 1), lambda r: (r, 0)),
            pl.BlockSpec((4, BR, Q), lambda r: (0, r, 0)),
        ],
        out_shape=[
            jax.ShapeDtypeStruct((N, 1), jnp.float32),
            jax.ShapeDtypeStruct((4, N, Q), jnp.float32),
        ],
    )(x, W1, degA, degB)


def _tc_mid_body(s_ref, p_ref, dinv_ref, b_ref, gs_ref, be_ref,
                 w_ref, o_ref):
    sc = _cat_quarters(s_ref)
    pc = jnp.concatenate([p_ref[q] for q in range(4)], axis=1)
    d = dinv_ref[...]
    x2 = jnp.maximum((d * (sc + pc) + b_ref[...]) * gs_ref[...] + be_ref[...], 0.0)
    h = jnp.dot(x2, w_ref[...], preferred_element_type=jnp.float32)
    _store_quarters(o_ref, h * d)


def _tc_mid(s1, p1, dinv, b, gs, be, W):
    return pl.pallas_call(
        _tc_mid_body,
        grid=(N // BR,),
        in_specs=[
            pl.BlockSpec((4, BR, Q), lambda r: (0, r, 0)),
            pl.BlockSpec((4, BR, Q), lambda r: (0, r, 0)),
            pl.BlockSpec((BR, 1), lambda r: (r, 0)),
            pl.BlockSpec((1, H), lambda r: (0, 0)),
            pl.BlockSpec((1, H), lambda r: (0, 0)),
            pl.BlockSpec((1, H), lambda r: (0, 0)),
            pl.BlockSpec((H, H), lambda r: (0, 0)),
        ],
        out_specs=pl.BlockSpec((4, BR, Q), lambda r: (0, r, 0)),
        out_shape=jax.ShapeDtypeStruct((4, N, Q), jnp.float32),
    )(s1, p1, dinv, b, gs, be, W)


def _tc3_body(s_ref, p_ref, dinv_ref, b_ref, gs_ref, be_ref,
              w_ref, o_ref):
    sc = _cat_quarters(s_ref)
    pc = jnp.concatenate([p_ref[q] for q in range(4)], axis=1)
    d = dinv_ref[...]
    x3 = jnp.maximum((d * (sc + pc) + b_ref[...]) * gs_ref[...] + be_ref[...], 0.0)
    h = jnp.dot(x3, w_ref[...], preferred_element_type=jnp.float32)
    o_ref[...] = h * d


def _tc3(s2, p2, dinv, b, gs, be, Wp):
    return pl.pallas_call(
        _tc3_body,
        grid=(N // BR,),
        in_specs=[
            pl.BlockSpec((4, BR, Q), lambda r: (0, r, 0)),
            pl.BlockSpec((4, BR, Q), lambda r: (0, r, 0)),
            pl.BlockSpec((BR, 1), lambda r: (r, 0)),
            pl.BlockSpec((1, H), lambda r: (0, 0)),
            pl.BlockSpec((1, H), lambda r: (0, 0)),
            pl.BlockSpec((1, H), lambda r: (0, 0)),
            pl.BlockSpec((H, Q), lambda r: (0, 0)),
        ],
        out_specs=pl.BlockSpec((BR, Q), lambda r: (r, 0)),
        out_shape=jax.ShapeDtypeStruct((N, Q), jnp.float32),
    )(s2, p2, dinv, b, gs, be, Wp)


def _tc4_body(s_ref, p_ref, dinv_ref, b_ref, o_ref):
    d = dinv_ref[...]
    l = d * (s_ref[0] + s_ref[1] + p_ref[...]) + b_ref[...]
    col = lax.broadcasted_iota(jnp.int32, (BR, Q), 1)
    valid = col < C
    lm = jnp.where(valid, l, jnp.float32(-1e30))
    m = jnp.max(lm, axis=1, keepdims=True)
    e = jnp.where(valid, jnp.exp(l - m), 0.0)
    lse = jnp.log(jnp.sum(e, axis=1, keepdims=True))
    o_ref[...] = (l - m - lse)[:, :C]


def _tc4(s3, p3, dinv, b3p):
    return pl.pallas_call(
        _tc4_body,
        grid=(N // BR,),
        in_specs=[
            pl.BlockSpec((2, BR, Q), lambda r: (0, r, 0)),
            pl.BlockSpec((BR, Q), lambda r: (r, 0)),
            pl.BlockSpec((BR, 1), lambda r: (r, 0)),
            pl.BlockSpec((1, Q), lambda r: (0, 0)),
        ],
        out_specs=pl.BlockSpec((BR, C), lambda r: (r, 0)),
        out_shape=jax.ShapeDtypeStruct((N, C), jnp.float32),
    )(s3, p3, dinv, b3p)


def kernel(x, edge_index, W1, b1, g1, be1, W2, b2, g2, be2, W3, b3):
    src = edge_index[0]
    dst = edge_index[1]
    scale = jnp.float32(1.0) / jnp.sqrt(jnp.float32(1.0 + 1e-5))
    gs1 = (g1 * scale).reshape(1, H)
    gs2 = (g2 * scale).reshape(1, H)
    b1r = b1.reshape(1, H)
    b2r = b2.reshape(1, H)
    be1r = be1.reshape(1, H)
    be2r = be2.reshape(1, H)
    W3p = jnp.pad(W3, ((0, 0), (0, Q - C)))
    b3p = jnp.pad(b3, (0, Q - C)).reshape(1, Q)

    s16 = jnp.pad(src.reshape(NS, E // NS), ((0, 0), (0, CH12 * B - E // NS)))
    d16 = jnp.pad(dst.reshape(NS, E // NS), ((0, 0), (0, CH12 * B - E // NS)),
                  constant_values=N)
    s16 = s16.reshape(NS, CH12, B)
    d16 = d16.reshape(NS, CH12, B)
    srcQ = [jnp.concatenate([s16 + (2 * k) * N, s16 + (2 * k + 1) * N], axis=0)
            for k in range(2)]
    dstA = jnp.concatenate([d16, d16], axis=0)
    src3 = jnp.pad(src.reshape(NW, E // NW), ((0, 0), (0, CH3 * B - E // NW))
                   ).reshape(NW, CH3, B)
    dst3 = jnp.pad(dst.reshape(NW, E // NW), ((0, 0), (0, CH3 * B - E // NW)),
                   constant_values=N).reshape(NW, CH3, B)

    deg = _deg_kernel(dst3)
    degA = deg[:N].reshape(N, 1)
    degB = deg[NPAD:NPAD + N].reshape(N, 1)

    dinv, p1 = _tc1(x, W1, degA, degB)
    p1f = p1.reshape(4 * N, Q)
    s1 = _agg12(p1f, srcQ[0], srcQ[1], dstA).reshape(4, NPAD, Q)
    p2 = _tc_mid(s1, p1, dinv, b1r, gs1, be1r, W2)
    p2f = p2.reshape(4 * N, Q)
    s2 = _agg12(p2f, srcQ[0], srcQ[1], dstA).reshape(4, NPAD, Q)
    p3 = _tc3(s2, p2, dinv, b2r, gs2, be2r, W3p)
    s3 = _agg3(p3, src3, dst3).reshape(2, NPAD, Q)
    return _tc4(s3, p3, dinv, b3p)


# speedup vs baseline: 1.0051x; 1.0051x over previous
import functools

import jax
import jax.numpy as jnp
from jax import lax
from jax.experimental import pallas as pl
from jax.experimental.pallas import tpu as pltpu
from jax.experimental.pallas import tpu_sc as plsc

N = 10000
E = 160000
D = 256
H = 256
C = 40
Q = 64

NC, NS = 2, 16
NW = NC * NS
NPAD = 10240
RPT = NPAD // NS
B = 128
CH12 = 80
CH3 = 40
BR = 2000

_SC_PARAMS = pltpu.CompilerParams(needs_layout_passes=False,
                                  use_tc_tiling_on_sc=False)


def _sc_mesh():
    return plsc.VectorSubcoreMesh(core_axis_name="c", subcore_axis_name="s")


@functools.partial(
    pl.kernel,
    out_type=jax.ShapeDtypeStruct((NC * NPAD,), jnp.float32),
    mesh=_sc_mesh(),
    compiler_params=_SC_PARAMS,
    scratch_types=[
        pltpu.VMEM((CH3, B), jnp.int32),
        pltpu.VMEM((NPAD,), jnp.float32),
        pltpu.VMEM((RPT,), jnp.float32),
        pltpu.VMEM((RPT,), jnp.float32),
        pltpu.VMEM_SHARED((NS * NPAD,), jnp.float32),
    ],
)
def _deg_kernel(dst_hbm, deg_out, slab, deg_v, tmp_v, acc_v, parts):
    c = lax.axis_index("c")
    s = lax.axis_index("s")
    w = c * NS + s
    zero16 = jnp.zeros((16,), jnp.float32)

    def zbody(i, _):
        deg_v[pl.ds(i * 16, 16)] = zero16
        return 0
    lax.fori_loop(0, NPAD // 16, zbody, 0)

    pltpu.sync_copy(dst_hbm.at[w], slab)
    ones = jnp.ones((16,), jnp.float32)

    def body(i, _):
        for k in range(B // 16):
            idx = slab[i, pl.ds(k * 16, 16)]
            plsc.addupdate_scatter(deg_v, [idx], ones)
        return 0
    lax.fori_loop(0, CH3, body, 0)

    pltpu.sync_copy(deg_v, parts.at[pl.ds(s * NPAD, NPAD)])
    plsc.subcore_barrier()

    def zbody2(i, _):
        acc_v[pl.ds(i * 16, 16)] = zero16
        return 0
    lax.fori_loop(0, RPT // 16, zbody2, 0)

    def rbody(j, _):
        pltpu.sync_copy(parts.at[pl.ds(j * NPAD + s * RPT, RPT)], tmp_v)

        def abody(k, _):
            acc_v[pl.ds(k * 16, 16)] = acc_v[pl.ds(k * 16, 16)] + tmp_v[pl.ds(k * 16, 16)]
            return 0
        lax.fori_loop(0, RPT // 16, abody, 0)
        return 0
    lax.fori_loop(0, NS, rbody, 0)

    pltpu.sync_copy(acc_v, deg_out.at[pl.ds(c * NPAD + s * RPT, RPT)])


def _make_agg(CH, npass):
    @functools.partial(
        pl.kernel,
        out_type=jax.ShapeDtypeStruct((2 * npass * NPAD, Q), jnp.float32),
        mesh=_sc_mesh(),
        compiler_params=_SC_PARAMS,
        scratch_types=[
            [pltpu.VMEM((CH, B), jnp.int32) for _ in range(npass)],
            pltpu.VMEM((CH, B), jnp.int32),
            [pltpu.VMEM((B, Q), jnp.float32) for _ in range(4)],
            pltpu.VMEM((B, Q), jnp.float32),
            pltpu.VMEM_SHARED((NPAD, Q), jnp.float32),
            [pltpu.SemaphoreType.DMA for _ in range(4)],
            [pltpu.SemaphoreType.DMA for _ in range(4)],
        ],
    )
    def agg(p_hbm, *args):
        src_hbms = args[:npass]
        dst_hbm = args[npass]
        out_hbm = args[npass + 1]
        src_vs = args[npass + 2]
        dst_v, rows, zb, acc, gsem, ssem = args[npass + 3:npass + 9]
        c = lax.axis_index("c")
        s = lax.axis_index("s")
        w = c * NS + s
        dw = w if npass == 1 else s
        zero16 = jnp.zeros((16,), jnp.float32)

        def zrow(i, _):
            for k in range(Q // 16):
                zb[i, pl.ds(k * 16, 16)] = zero16
            return 0
        lax.fori_loop(0, B, zrow, 0)

        for k in range(npass):
            pltpu.sync_copy(src_hbms[k].at[w], src_vs[k])
        pltpu.sync_copy(dst_hbm.at[dw], dst_v)

        for k in range(npass):
            src_v = src_vs[k]

            def zacc(j, _):
                pltpu.sync_copy(zb, acc.at[pl.ds(s * RPT + j * B, B)])
                return 0
            lax.fori_loop(0, RPT // B, zacc, 0)
            plsc.subcore_barrier()

            G = CH // 4
            for b in range(4):
                pltpu.async_copy(p_hbm.at[src_v.at[b]], rows[b], gsem[b])

            def step(g, _):
                ch = 4 * g
                for b in range(4):
                    pltpu.make_async_copy(p_hbm.at[src_v.at[ch + b]],
                                          rows[b], gsem[b]).wait()
                    pltpu.async_copy(rows[b], acc.at[dst_v.at[ch + b]],
                                     ssem[b], add=True)

                @pl.when(g < G - 1)
                def _():
                    for b in range(4):
                        pltpu.make_async_copy(rows[b],
                                              acc.at[dst_v.at[ch + b]],
                                              ssem[b]).wait()
                        pltpu.async_copy(p_hbm.at[src_v.at[ch + 4 + b]],
                                         rows[b], gsem[b])
                return 0
            lax.fori_loop(0, G, step, 0)

            for b in range(4):
                pltpu.make_async_copy(rows[b], acc.at[dst_v.at[CH - 4 + b]],
                                      ssem[b]).wait()

            plsc.subcore_barrier()
            qidx = (2 * k + c) if npass > 1 else c
            pltpu.sync_copy(
                acc.at[pl.ds(s * RPT, RPT)],
                out_hbm.at[pl.ds(qidx * NPAD + s * RPT, RPT)])
            if k + 1 < npass:
                plsc.subcore_barrier()
    return agg


_agg12 = _make_agg(CH12, 2)
_agg3 = _make_agg(CH3, 1)


def _store_quarters(o_ref, p):
    for q in range(4):
        o_ref[q, :, :] = p[:, q * Q:(q + 1) * Q]


def _cat_quarters(s_ref):
    return jnp.concatenate([s_ref[q] for q in range(4)], axis=1)


def _tc1_body(x_ref, w_ref, da_ref, db_ref, dinv_ref, p_ref):
    d = lax.rsqrt(da_ref[...] + db_ref[...] + 1.0)
    dinv_ref[...] = d
    h = jnp.dot(x_ref[...], w_ref[...], preferred_element_type=jnp.float32)
    _store_quarters(p_ref, h * d)


def _tc1(x, W1, degA, degB):
    return pl.pallas_call(
        _tc1_body,
        grid=(N // BR,),
        in_specs=[
            pl.BlockSpec((BR, D), lambda r: (r, 0)),
            pl.BlockSpec((D, H), lambda r: (0, 0)),
            pl.BlockSpec((BR, 1), lambda r: (r, 0)),
            pl.BlockSpec((BR, 1), lambda r: (r, 0)),
        ],
        out_specs=[
            pl.BlockSpec((BR, 1), lambda r: (r, 0)),
            pl.BlockSpec((4, BR, Q), lambda r: (0, r, 0)),
        ],
        out_shape=[
            jax.ShapeDtypeStruct((N, 1), jnp.float32),
            jax.ShapeDtypeStruct((4, N, Q), jnp.float32),
        ],
    )(x, W1, degA, degB)


def _tc_mid_body(s_ref, p_ref, dinv_ref, b_ref, gs_ref, be_ref,
                 w_ref, o_ref):
    sc = _cat_quarters(s_ref)
    pc = jnp.concatenate([p_ref[q] for q in range(4)], axis=1)
    d = dinv_ref[...]
    x2 = jnp.maximum((d * (sc + pc) + b_ref[...]) * gs_ref[...] + be_ref[...], 0.0)
    h = jnp.dot(x2, w_ref[...], preferred_element_type=jnp.float32)
    _store_quarters(o_ref, h * d)


def _tc_mid(s1, p1, dinv, b, gs, be, W):
    return pl.pallas_call(
        _tc_mid_body,
        grid=(N // BR,),
        in_specs=[
            pl.BlockSpec((4, BR, Q), lambda r: (0, r, 0)),
            pl.BlockSpec((4, BR, Q), lambda r: (0, r, 0)),
            pl.BlockSpec((BR, 1), lambda r: (r, 0)),
            pl.BlockSpec((1, H), lambda r: (0, 0)),
            pl.BlockSpec((1, H), lambda r: (0, 0)),
            pl.BlockSpec((1, H), lambda r: (0, 0)),
            pl.BlockSpec((H, H), lambda r: (0, 0)),
        ],
        out_specs=pl.BlockSpec((4, BR, Q), lambda r: (0, r, 0)),
        out_shape=jax.ShapeDtypeStruct((4, N, Q), jnp.float32),
    )(s1, p1, dinv, b, gs, be, W)


def _tc3_body(s_ref, p_ref, dinv_ref, b_ref, gs_ref, be_ref,
              w_ref, o_ref):
    sc = _cat_quarters(s_ref)
    pc = jnp.concatenate([p_ref[q] for q in range(4)], axis=1)
    d = dinv_ref[...]
    x3 = jnp.maximum((d * (sc + pc) + b_ref[...]) * gs_ref[...] + be_ref[...], 0.0)
    h = jnp.dot(x3, w_ref[...], preferred_element_type=jnp.float32)
    o_ref[...] = h * d


def _tc3(s2, p2, dinv, b, gs, be, Wp):
    return pl.pallas_call(
        _tc3_body,
        grid=(N // BR,),
        in_specs=[
            pl.BlockSpec((4, BR, Q), lambda r: (0, r, 0)),
            pl.BlockSpec((4, BR, Q), lambda r: (0, r, 0)),
            pl.BlockSpec((BR, 1), lambda r: (r, 0)),
            pl.BlockSpec((1, H), lambda r: (0, 0)),
            pl.BlockSpec((1, H), lambda r: (0, 0)),
            pl.BlockSpec((1, H), lambda r: (0, 0)),
            pl.BlockSpec((H, Q), lambda r: (0, 0)),
        ],
        out_specs=pl.BlockSpec((BR, Q), lambda r: (r, 0)),
        out_shape=jax.ShapeDtypeStruct((N, Q), jnp.float32),
    )(s2, p2, dinv, b, gs, be, Wp)


def _tc4_body(s_ref, p_ref, dinv_ref, b_ref, o_ref):
    d = dinv_ref[...]
    l = d * (s_ref[0] + s_ref[1] + p_ref[...]) + b_ref[...]
    col = lax.broadcasted_iota(jnp.int32, (BR, Q), 1)
    valid = col < C
    lm = jnp.where(valid, l, jnp.float32(-1e30))
    m = jnp.max(lm, axis=1, keepdims=True)
    e = jnp.where(valid, jnp.exp(l - m), 0.0)
    lse = jnp.log(jnp.sum(e, axis=1, keepdims=True))
    o_ref[...] = (l - m - lse)[:, :C]


def _tc4(s3, p3, dinv, b3p):
    return pl.pallas_call(
        _tc4_body,
        grid=(N // BR,),
        in_specs=[
            pl.BlockSpec((2, BR, Q), lambda r: (0, r, 0)),
            pl.BlockSpec((BR, Q), lambda r: (r, 0)),
            pl.BlockSpec((BR, 1), lambda r: (r, 0)),
            pl.BlockSpec((1, Q), lambda r: (0, 0)),
        ],
        out_specs=pl.BlockSpec((BR, C), lambda r: (r, 0)),
        out_shape=jax.ShapeDtypeStruct((N, C), jnp.float32),
    )(s3, p3, dinv, b3p)


def kernel(x, edge_index, W1, b1, g1, be1, W2, b2, g2, be2, W3, b3):
    src = edge_index[0]
    dst = edge_index[1]
    scale = jnp.float32(1.0) / jnp.sqrt(jnp.float32(1.0 + 1e-5))
    gs1 = (g1 * scale).reshape(1, H)
    gs2 = (g2 * scale).reshape(1, H)
    b1r = b1.reshape(1, H)
    b2r = b2.reshape(1, H)
    be1r = be1.reshape(1, H)
    be2r = be2.reshape(1, H)
    W3p = jnp.pad(W3, ((0, 0), (0, Q - C)))
    b3p = jnp.pad(b3, (0, Q - C)).reshape(1, Q)

    s16 = jnp.pad(src.reshape(NS, E // NS), ((0, 0), (0, CH12 * B - E // NS))
                  ).reshape(NS, CH12, B)
    d16 = jnp.pad(dst.reshape(NS, E // NS), ((0, 0), (0, CH12 * B - E // NS)),
                  constant_values=N).reshape(NS, CH12, B)
    srcQ = [jnp.concatenate([s16 + (2 * k) * N, s16 + (2 * k + 1) * N], axis=0)
            for k in range(2)]
    src3 = s16.reshape(NW, CH3, B)
    dst3 = d16.reshape(NW, CH3, B)

    deg = _deg_kernel(dst3)
    degA = deg[:N].reshape(N, 1)
    degB = deg[NPAD:NPAD + N].reshape(N, 1)

    dinv, p1 = _tc1(x, W1, degA, degB)
    p1f = p1.reshape(4 * N, Q)
    s1 = _agg12(p1f, srcQ[0], srcQ[1], d16).reshape(4, NPAD, Q)
    p2 = _tc_mid(s1, p1, dinv, b1r, gs1, be1r, W2)
    p2f = p2.reshape(4 * N, Q)
    s2 = _agg12(p2f, srcQ[0], srcQ[1], d16).reshape(4, NPAD, Q)
    p3 = _tc3(s2, p2, dinv, b2r, gs2, be2r, W3p)
    s3 = _agg3(p3, src3, dst3).reshape(2, NPAD, Q)
    return _tc4(s3, p3, dinv, b3p)
